# baseline (speedup 1.0000x reference)
import functools

import jax
import jax.numpy as jnp
from jax.experimental import pallas as pl
from jax.experimental.pallas import tpu as pltpu


def _body(x1_ref, x2_ref, p1_ref, p2_ref, w0_ref, b0_ref, w1_ref, b1_ref,
          g0_ref, be0_ref, g1_ref, be1_ref, out_ref,
          y0s, y1s, st0, st1, *, S, Nb, C0, C1, count):
    p = pl.program_id(0)
    b = pl.program_id(1)
    n = pl.program_id(2)

    @pl.when(p == 0)
    def _phase0():
        x1 = x1_ref[0]
        x2 = x2_ref[0]
        g = jax.lax.dot_general(x2, x1, (((0,), (0,)), ((), ())),
                                preferred_element_type=jnp.float32)
        distT = 0.5 * jnp.sum(x2 * x2, axis=0)[:, None] - g

        d012 = 2.0 * distT[0:3, :] + jnp.sum(x1 * x1, axis=0)[None, :]
        d012 = jnp.where(d012 < 1e-10, 1e-10, d012)
        w = 1.0 / d012
        w = w / jnp.sum(w, axis=0, keepdims=True)

        inf = jnp.full((8, Nb), jnp.inf, dtype=jnp.float32)
        m1, m2, m3 = inf, inf, inf
        for c in range(S // 8):
            v = distT[8 * c:8 * (c + 1), :]
            hi1 = jnp.maximum(m1, v)
            m1 = jnp.minimum(m1, v)
            hi2 = jnp.maximum(m2, hi1)
            m2 = jnp.minimum(m2, hi1)
            m3 = jnp.minimum(m3, hi2)

        iota8 = jax.lax.broadcasted_iota(jnp.int32, (8, Nb), 0)
        k = jnp.zeros((8, Nb), dtype=jnp.int32)
        vals = []
        cand = m1
        for _ in range(3):
            vk = jnp.min(cand, axis=0, keepdims=True)
            vals.append(vk)
            rk = jnp.min(jnp.where(cand == vk, iota8, 8), axis=0,
                         keepdims=True)
            k = k + jnp.where(iota8 == rk, 1, 0)
            cand = jnp.where(k == 0, m1, jnp.where(k == 1, m2, m3))
        v1, v2, v3 = vals

        at = jnp.where(
            distT == v1, w[0:1, :],
            jnp.where(distT == v2, w[1:2, :],
                      jnp.where(distT == v3, w[2:3, :], 0.0)))

        interpT = jnp.dot(p2_ref[0], at, preferred_element_type=jnp.float32)

        p1 = p1_ref[0]
        W0 = w0_ref[...]
        D1 = p1.shape[0]
        y = (jnp.dot(W0[:, :D1], p1, preferred_element_type=jnp.float32)
             + jnp.dot(W0[:, D1:], interpT,
                       preferred_element_type=jnp.float32)
             + b0_ref[...])
        y0s[pl.ds(b * C0, C0), pl.ds(n * Nb, Nb)] = y.astype(jnp.bfloat16)

        s1 = jnp.sum(y, axis=1)[None, :]
        s2 = jnp.sum(y * y, axis=1)[None, :]
        upd = jnp.concatenate(
            [s1, s2, jnp.zeros((6, C0), jnp.float32)], axis=0)

        @pl.when(jnp.logical_and(b == 0, n == 0))
        def _():
            st0[...] = jnp.zeros_like(st0)

        st0[...] += upd

    @pl.when(p == 1)
    def _phase1():
        mean = st0[0, :] / count
        var = st0[1, :] / count - mean * mean
        scale = g0_ref[0] * jax.lax.rsqrt(var + 1e-5)
        shift = be0_ref[0] - mean * scale
        y0 = y0s[pl.ds(b * C0, C0), pl.ds(n * Nb, Nb)].astype(jnp.float32)
        h = jnp.maximum(y0 * scale[:, None] + shift[:, None], 0.0)
        y = (jnp.dot(w1_ref[...], h, preferred_element_type=jnp.float32)
             + b1_ref[...])
        y1s[pl.ds(b * C1, C1), pl.ds(n * Nb, Nb)] = y.astype(jnp.bfloat16)

        s1 = jnp.sum(y, axis=1)[None, :]
        s2 = jnp.sum(y * y, axis=1)[None, :]
        upd = jnp.concatenate(
            [s1, s2, jnp.zeros((6, C1), jnp.float32)], axis=0)

        @pl.when(jnp.logical_and(b == 0, n == 0))
        def _():
            st1[...] = jnp.zeros_like(st1)

        st1[...] += upd

    @pl.when(p == 2)
    def _phase2():
        mean = st1[0, :] / count
        var = st1[1, :] / count - mean * mean
        scale = g1_ref[0] * jax.lax.rsqrt(var + 1e-5)
        shift = be1_ref[0] - mean * scale
        y1 = y1s[pl.ds(b * C1, C1), pl.ds(n * Nb, Nb)].astype(jnp.float32)
        out_ref[0] = jnp.maximum(y1 * scale[:, None] + shift[:, None], 0.0)


@jax.jit
def kernel(xyz1, xyz2, points1, points2, W0, b0, gamma0, beta0,
           W1, b1, gamma1, beta1):
    B, _, N = xyz1.shape
    S = xyz2.shape[2]
    D1 = points1.shape[1]
    D2 = points2.shape[1]
    C0 = W0.shape[0]
    C1 = W1.shape[0]
    Nb = 1024
    grid = (3, B, N // Nb)
    count = float(B * N)

    b0c = b0.reshape(C0, 1)
    b1c = b1.reshape(C1, 1)
    g0r = gamma0.reshape(1, C0)
    be0r = beta0.reshape(1, C0)
    g1r = gamma1.reshape(1, C1)
    be1r = beta1.reshape(1, C1)

    def _p0(p, b, n):
        return (jnp.where(p == 0, b, 0), 0, jnp.where(p == 0, n, 0))

    def _pb(p, b, n):
        return (jnp.where(p == 0, b, 0), 0, 0)

    out = pl.pallas_call(
        functools.partial(_body, S=S, Nb=Nb, C0=C0, C1=C1, count=count),
        grid=grid,
        in_specs=[
            pl.BlockSpec((1, 3, Nb), _p0),
            pl.BlockSpec((1, 3, S), _pb),
            pl.BlockSpec((1, D1, Nb), _p0),
            pl.BlockSpec((1, D2, S), _pb),
            pl.BlockSpec((C0, D1 + D2), lambda p, b, n: (0, 0)),
            pl.BlockSpec((C0, 1), lambda p, b, n: (0, 0)),
            pl.BlockSpec((C1, C0), lambda p, b, n: (0, 0)),
            pl.BlockSpec((C1, 1), lambda p, b, n: (0, 0)),
            pl.BlockSpec((1, C0), lambda p, b, n: (0, 0)),
            pl.BlockSpec((1, C0), lambda p, b, n: (0, 0)),
            pl.BlockSpec((1, C1), lambda p, b, n: (0, 0)),
            pl.BlockSpec((1, C1), lambda p, b, n: (0, 0)),
        ],
        out_specs=pl.BlockSpec((1, C1, Nb), lambda p, b, n:
                               (jnp.where(p == 2, b, 0), 0,
                                jnp.where(p == 2, n, 0))),
        out_shape=jax.ShapeDtypeStruct((B, C1, N), jnp.float32),
        scratch_shapes=[
            pltpu.VMEM((B * C0, N), jnp.bfloat16),
            pltpu.VMEM((B * C1, N), jnp.bfloat16),
            pltpu.VMEM((8, C0), jnp.float32),
            pltpu.VMEM((8, C1), jnp.float32),
        ],
    )(xyz1, xyz2, points1, points2, W0, b0c, W1, b1c, g0r, be0r, g1r, be1r)

    return out


# speedup vs baseline: 1.1600x; 1.1600x over previous
import functools

import jax
import jax.numpy as jnp
from jax.experimental import pallas as pl


def _k1_body(x1_ref, x2_ref, p1_ref, p2_ref, w0_ref, b0_ref, y0_ref, st_ref,
             *, S, Nb):
    b = pl.program_id(0)
    n = pl.program_id(1)

    x1 = x1_ref[0]
    x2 = x2_ref[0]
    g = jax.lax.dot_general(x2, x1, (((0,), (0,)), ((), ())),
                            preferred_element_type=jnp.float32)
    distT = 0.5 * jnp.sum(x2 * x2, axis=0)[:, None] - g

    d012 = 2.0 * distT[0:3, :] + jnp.sum(x1 * x1, axis=0)[None, :]
    d012 = jnp.where(d012 < 1e-10, 1e-10, d012)
    w = 1.0 / d012
    w = w / jnp.sum(w, axis=0, keepdims=True)

    inf = jnp.full((8, Nb), jnp.inf, dtype=jnp.float32)
    m1, m2, m3 = inf, inf, inf
    for c in range(S // 8):
        v = distT[8 * c:8 * (c + 1), :]
        hi1 = jnp.maximum(m1, v)
        m1 = jnp.minimum(m1, v)
        hi2 = jnp.maximum(m2, hi1)
        m2 = jnp.minimum(m2, hi1)
        m3 = jnp.minimum(m3, hi2)

    iota8 = jax.lax.broadcasted_iota(jnp.int32, (8, Nb), 0)
    k = jnp.zeros((8, Nb), dtype=jnp.int32)
    vals = []
    cand = m1
    for _ in range(3):
        vk = jnp.min(cand, axis=0, keepdims=True)
        vals.append(vk)
        rk = jnp.min(jnp.where(cand == vk, iota8, 8), axis=0,
                     keepdims=True)
        k = k + jnp.where(iota8 == rk, 1, 0)
        cand = jnp.where(k == 0, m1, jnp.where(k == 1, m2, m3))
    v1, v2, v3 = vals

    at = jnp.where(
        distT == v1, w[0:1, :],
        jnp.where(distT == v2, w[1:2, :],
                  jnp.where(distT == v3, w[2:3, :], 0.0)))

    interpT = jnp.dot(p2_ref[0], at, preferred_element_type=jnp.float32)

    p1 = p1_ref[0]
    W0 = w0_ref[...]
    D1 = p1.shape[0]
    y = (jnp.dot(W0[:, :D1], p1, preferred_element_type=jnp.float32)
         + jnp.dot(W0[:, D1:], interpT, preferred_element_type=jnp.float32)
         + b0_ref[...])
    y0_ref[0] = y.astype(jnp.bfloat16)

    s1 = jnp.sum(y, axis=1)[None, :]
    s2 = jnp.sum(y * y, axis=1)[None, :]
    upd = jnp.concatenate(
        [s1, s2, jnp.zeros((6, s1.shape[1]), jnp.float32)], axis=0)

    @pl.when(jnp.logical_and(b == 0, n == 0))
    def _():
        st_ref[...] = jnp.zeros_like(st_ref)

    st_ref[...] += upd


def _k2_body(y0_ref, st_ref, g0_ref, be0_ref, w1_ref, b1_ref, y1_ref,
             st1_ref, *, count):
    b = pl.program_id(0)
    n = pl.program_id(1)

    mean = st_ref[0, :] / count
    var = st_ref[1, :] / count - mean * mean
    scale = g0_ref[0] * jax.lax.rsqrt(var + 1e-5)
    shift = be0_ref[0] - mean * scale
    h = jnp.maximum(y0_ref[0].astype(jnp.float32) * scale[:, None]
                    + shift[:, None], 0.0)
    y = (jnp.dot(w1_ref[...], h, preferred_element_type=jnp.float32)
         + b1_ref[...])
    y1_ref[0] = y.astype(jnp.bfloat16)

    s1 = jnp.sum(y, axis=1)[None, :]
    s2 = jnp.sum(y * y, axis=1)[None, :]
    upd = jnp.concatenate(
        [s1, s2, jnp.zeros((6, s1.shape[1]), jnp.float32)], axis=0)

    @pl.when(jnp.logical_and(b == 0, n == 0))
    def _():
        st1_ref[...] = jnp.zeros_like(st1_ref)

    st1_ref[...] += upd


def _k3_body(y1_ref, st_ref, g1_ref, be1_ref, out_ref, *, count):
    mean = st_ref[0, :] / count
    var = st_ref[1, :] / count - mean * mean
    scale = g1_ref[0] * jax.lax.rsqrt(var + 1e-5)
    shift = be1_ref[0] - mean * scale
    out_ref[0] = jnp.maximum(y1_ref[0].astype(jnp.float32) * scale[:, None]
                             + shift[:, None], 0.0)


@jax.jit
def kernel(xyz1, xyz2, points1, points2, W0, b0, gamma0, beta0,
           W1, b1, gamma1, beta1):
    B, _, N = xyz1.shape
    S = xyz2.shape[2]
    D1 = points1.shape[1]
    D2 = points2.shape[1]
    C0 = W0.shape[0]
    C1 = W1.shape[0]
    Nb = 4096
    grid = (B, N // Nb)
    Nb2 = 4096
    grid2 = (B, N // Nb2)
    count = float(B * N)

    b0c = b0.reshape(C0, 1)
    b1c = b1.reshape(C1, 1)
    g0r = gamma0.reshape(1, C0)
    be0r = beta0.reshape(1, C0)
    g1r = gamma1.reshape(1, C1)
    be1r = beta1.reshape(1, C1)

    y0, st0 = pl.pallas_call(
        functools.partial(_k1_body, S=S, Nb=Nb),
        grid=grid,
        in_specs=[
            pl.BlockSpec((1, 3, Nb), lambda b, n: (b, 0, n)),
            pl.BlockSpec((1, 3, S), lambda b, n: (b, 0, 0)),
            pl.BlockSpec((1, D1, Nb), lambda b, n: (b, 0, n)),
            pl.BlockSpec((1, D2, S), lambda b, n: (b, 0, 0)),
            pl.BlockSpec((C0, D1 + D2), lambda b, n: (0, 0)),
            pl.BlockSpec((C0, 1), lambda b, n: (0, 0)),
        ],
        out_specs=[
            pl.BlockSpec((1, C0, Nb), lambda b, n: (b, 0, n)),
            pl.BlockSpec((8, C0), lambda b, n: (0, 0)),
        ],
        out_shape=[
            jax.ShapeDtypeStruct((B, C0, N), jnp.bfloat16),
            jax.ShapeDtypeStruct((8, C0), jnp.float32),
        ],
    )(xyz1, xyz2, points1, points2, W0, b0c)

    y1, st1 = pl.pallas_call(
        functools.partial(_k2_body, count=count),
        grid=grid2,
        in_specs=[
            pl.BlockSpec((1, C0, Nb2), lambda b, n: (b, 0, n)),
            pl.BlockSpec((8, C0), lambda b, n: (0, 0)),
            pl.BlockSpec((1, C0), lambda b, n: (0, 0)),
            pl.BlockSpec((1, C0), lambda b, n: (0, 0)),
            pl.BlockSpec((C1, C0), lambda b, n: (0, 0)),
            pl.BlockSpec((C1, 1), lambda b, n: (0, 0)),
        ],
        out_specs=[
            pl.BlockSpec((1, C1, Nb2), lambda b, n: (b, 0, n)),
            pl.BlockSpec((8, C1), lambda b, n: (0, 0)),
        ],
        out_shape=[
            jax.ShapeDtypeStruct((B, C1, N), jnp.bfloat16),
            jax.ShapeDtypeStruct((8, C1), jnp.float32),
        ],
    )(y0, st0, g0r, be0r, W1, b1c)

    out = pl.pallas_call(
        functools.partial(_k3_body, count=count),
        grid=grid2,
        in_specs=[
            pl.BlockSpec((1, C1, Nb2), lambda b, n: (b, 0, n)),
            pl.BlockSpec((8, C1), lambda b, n: (0, 0)),
            pl.BlockSpec((1, C1), lambda b, n: (0, 0)),
            pl.BlockSpec((1, C1), lambda b, n: (0, 0)),
        ],
        out_specs=pl.BlockSpec((1, C1, Nb2), lambda b, n: (b, 0, n)),
        out_shape=jax.ShapeDtypeStruct((B, C1, N), jnp.float32),
    )(y1, st1, g1r, be1r)

    return out
